# baseline (speedup 1.0000x reference)
import functools

import jax
import jax.numpy as jnp
from jax import lax
from jax.experimental import pallas as pl
from jax.experimental.pallas import tpu as pltpu
from jax.experimental.pallas import tpu_sc as plsc

N_NODES = 50000
N_EDGES = 800000
N_GRAPHS = 128
HID = 64

NPAD = 51200
EPAD = 819200
CHUNK = 128
NCH = EPAD // CHUNK
NB16, BID16 = 8, 40
NBP, BIDP = 3, 5
NHALF = NPAD // 2
CAPC = 205
CAPE = CAPC * CHUNK
PADC = BIDP * CHUNK
ACCR = NHALF + CHUNK
NC = 2
NS = 16
RS = NPAD // NS
BLK = 2048
NBLK = NPAD // BLK

_HI = jax.lax.Precision.DEFAULT

_sc_mesh = plsc.VectorSubcoreMesh(core_axis_name="c", subcore_axis_name="s")


IDB = 1600
EPT = EPAD // (NC * NS)


@functools.partial(
    pl.kernel,
    out_type=jax.ShapeDtypeStruct((NC, NPAD), jnp.float32),
    mesh=_sc_mesh,
    scratch_types=[
        pltpu.VMEM((IDB,), jnp.int32),
        pltpu.VMEM((NPAD,), jnp.float32),
        pltpu.VMEM_SHARED((NS, NPAD), jnp.float32),
        pltpu.VMEM((RS,), jnp.float32),
        pltpu.VMEM((RS,), jnp.float32),
    ],
    compiler_params=pltpu.CompilerParams(needs_layout_passes=False),
)
def _deg(dst_ref, out_ref, ids, degloc, part, tmp, acc):
    c = lax.axis_index("c")
    s = lax.axis_index("s")
    w = c * NS + s
    z16 = jnp.zeros((16,), jnp.float32)

    def zero_body(i, carry):
        degloc[pl.ds(i * 16, 16)] = z16
        return carry

    lax.fori_loop(0, NPAD // 16, zero_body, 0)

    ones = jnp.ones((16,), jnp.float32)
    base = w * EPT

    def blk_body(i, carry):
        pltpu.sync_copy(dst_ref.at[pl.ds(base + i * IDB, IDB)], ids)

        def inner(k, carry2):
            idx = ids[pl.ds(k * 16, 16)]
            plsc.addupdate_scatter(degloc, [idx], ones)
            return carry2

        lax.fori_loop(0, IDB // 16, inner, 0)
        return carry

    lax.fori_loop(0, EPT // IDB, blk_body, 0)

    pltpu.sync_copy(degloc, part.at[s])
    plsc.subcore_barrier()
    rb = s * RS
    pltpu.sync_copy(part.at[0, pl.ds(rb, RS)], acc)

    def red_body(t, carry):
        pltpu.sync_copy(part.at[t, pl.ds(rb, RS)], tmp)

        def vadd(i, carry2):
            acc[pl.ds(i * 16, 16)] = acc[pl.ds(i * 16, 16)] + tmp[pl.ds(i * 16, 16)]
            return carry2

        lax.fori_loop(0, RS // 16, vadd, 0)
        return carry

    lax.fori_loop(1, NS, red_body, 0)
    pltpu.sync_copy(acc, out_ref.at[c, pl.ds(rb, RS)])


NT = NPAD // (NC * NS)


def _rsqrt16(x):
    i = plsc.bitcast(x, jnp.int32)
    y = plsc.bitcast(jnp.int32(0x5F3759DF) - (i >> 1), jnp.float32)
    y = y * (1.5 - 0.5 * x * y * y)
    y = y * (1.5 - 0.5 * x * y * y)
    y = y * (1.5 - 0.5 * x * y * y)
    return y


@functools.partial(
    pl.kernel,
    out_type=[
        jax.ShapeDtypeStruct((NPAD,), jnp.float32),
        jax.ShapeDtypeStruct((NPAD * 16,), jnp.float32),
    ],
    mesh=_sc_mesh,
    scratch_types=[
        pltpu.VMEM((NT,), jnp.float32),
        pltpu.VMEM((NT,), jnp.float32),
        pltpu.VMEM((NT,), jnp.float32),
        pltpu.VMEM((NT * 16,), jnp.float32),
    ],
    compiler_params=pltpu.CompilerParams(use_tc_tiling_on_sc=False,
                                         needs_layout_passes=False),
)
def _scprep(degp_ref, xf_ref, dinv_ref, q1_ref, da, db, dloc, xbuf):
    c = lax.axis_index("c")
    s = lax.axis_index("s")
    nb = (c * NS + s) * NT
    pltpu.sync_copy(degp_ref.at[0, pl.ds(nb, NT)], da)
    pltpu.sync_copy(degp_ref.at[1, pl.ds(nb, NT)], db)
    pltpu.sync_copy(xf_ref.at[pl.ds(nb * 16, NT * 16)], xbuf)

    def vec_body(i, carry):
        d = da[pl.ds(i * 16, 16)] + db[pl.ds(i * 16, 16)] + 1.0
        dloc[pl.ds(i * 16, 16)] = _rsqrt16(d)
        return carry

    lax.fori_loop(0, NT // 16, vec_body, 0)
    pltpu.sync_copy(dloc, dinv_ref.at[pl.ds(nb, NT)])

    def row_body(i, carry):
        dv = dloc[pl.ds(i * 16, 16)]
        for k in range(16):
            base = (i * 16 + k) * 16
            xbuf[pl.ds(base, 16)] = xbuf[pl.ds(base, 16)] * dv[k]
        return carry

    lax.fori_loop(0, NT // 16, row_body, 0)
    pltpu.sync_copy(xbuf, q1_ref.at[pl.ds(nb * 16, NT * 16)])


def _edge_loop(src_ref, dst_ref, gather_ref, acc_sh, sidx, didx, rows, gsem,
               ssem, isem, c0, ntile, nb, bid):
    nblk = ntile // bid

    def load_ids(ib, blk):
        cb = c0 + blk * bid
        pltpu.async_copy(src_ref.at[pl.ds(cb, bid)], sidx.at[ib], isem.at[ib])
        pltpu.async_copy(dst_ref.at[pl.ds(cb, bid)], didx.at[ib], isem.at[ib])

    def wait_ids(ib):
        for r in (sidx, didx):
            pltpu.make_async_copy(src_ref.at[pl.ds(0, bid)], r.at[ib],
                                  isem.at[ib]).wait()

    def fire_gather(ib, u, b):
        pltpu.async_copy(gather_ref.at[sidx.at[ib, u]], rows.at[b],
                         gsem.at[b])

    def drain_gather(b):
        pltpu.make_async_copy(gather_ref.at[pl.ds(0, CHUNK)], rows.at[b],
                              gsem.at[b]).wait()

    def fire_add(ib, u, b):
        pltpu.async_copy(rows.at[b], acc_sh.at[didx.at[ib, u]], ssem.at[b],
                         add=True)

    def drain_add(b):
        pltpu.make_async_copy(rows.at[b], acc_sh.at[pl.ds(0, CHUNK)],
                              ssem.at[b]).wait()

    @pl.when(nblk >= 1)
    def _():
        load_ids(0, 0)

    def blk_body(blk, carry):
        ib = blk % 2
        wait_ids(ib)

        @pl.when(blk + 1 < nblk)
        def _():
            load_ids(1 - ib, blk + 1)

        for u in range(nb - 1):
            fire_gather(ib, u, u)
        for u in range(bid):
            b = u % nb
            drain_gather(b)
            fire_add(ib, u, b)
            nxt = u + nb - 1
            if nxt < bid:
                pb = nxt % nb
                if u >= 1:
                    drain_add(pb)
                fire_gather(ib, nxt, pb)
        for b in range(nb):
            drain_add(b)
        return carry

    lax.fori_loop(0, nblk, blk_body, 0)


@functools.partial(
    pl.kernel,
    out_type=jax.ShapeDtypeStruct((NC, NPAD, 16), jnp.float32),
    mesh=_sc_mesh,
    scratch_types=[
        pltpu.VMEM((2, BID16, CHUNK), jnp.int32),
        pltpu.VMEM((2, BID16, CHUNK), jnp.int32),
        pltpu.VMEM((NB16, CHUNK, 16), jnp.float32),
        pltpu.VMEM_SHARED((NPAD, 16), jnp.float32),
        pltpu.SemaphoreType.DMA((NB16,)),
        pltpu.SemaphoreType.DMA((NB16,)),
        pltpu.SemaphoreType.DMA((2,)),
    ],
    compiler_params=pltpu.CompilerParams(use_tc_tiling_on_sc=False),
)
def _prop16(src_ref, dst_ref, q_ref, z_ref, out_ref, sidx, didx, rows, acc_sh,
            gsem, ssem, isem):
    c = lax.axis_index("c")
    s = lax.axis_index("s")
    rb = s * RS

    @pl.when(c == 0)
    def _():
        pltpu.sync_copy(q_ref.at[pl.ds(rb, RS)], acc_sh.at[pl.ds(rb, RS)])

    @pl.when(c == 1)
    def _():
        pltpu.sync_copy(z_ref.at[pl.ds(rb, RS)], acc_sh.at[pl.ds(rb, RS)])

    plsc.subcore_barrier()
    per_core = NCH // NC
    per_tile = per_core // NS
    c0 = c * per_core + s * per_tile
    _edge_loop(src_ref, dst_ref, q_ref, acc_sh, sidx, didx, rows, gsem,
               ssem, isem, c0, per_tile, NB16, BID16)
    plsc.subcore_barrier()
    pltpu.sync_copy(acc_sh.at[pl.ds(rb, RS)], out_ref.at[c, pl.ds(rb, RS)])


@functools.partial(
    pl.kernel,
    out_type=[
        jax.ShapeDtypeStruct((2 * 32 * CAPE,), jnp.int32),
        jax.ShapeDtypeStruct((2 * 32 * CAPE,), jnp.int32),
        jax.ShapeDtypeStruct((2, 32, 16), jnp.int32),
    ],
    mesh=_sc_mesh,
    scratch_types=[
        pltpu.VMEM((IDB,), jnp.int32),
        pltpu.VMEM((IDB,), jnp.int32),
        pltpu.VMEM((CAPE + 16,), jnp.int32),
        pltpu.VMEM((CAPE + 16,), jnp.int32),
        pltpu.VMEM((CAPE + 16,), jnp.int32),
        pltpu.VMEM((CAPE + 16,), jnp.int32),
    ],
    compiler_params=pltpu.CompilerParams(use_tc_tiling_on_sc=False,
                                         needs_layout_passes=False),
)
def _bucket(src_ref, dst_ref, bsrc_ref, bdst_ref, cnt_ref, sbuf, dbuf,
            lsa, lda, lsb, ldb):
    c = lax.axis_index("c")
    s = lax.axis_index("s")
    w = c * NS + s
    base = w * EPT
    iota = lax.broadcasted_iota(jnp.int32, (16,), 0)

    def blk_body(i, offs):
        pltpu.sync_copy(src_ref.at[pl.ds(base + i * IDB, IDB)], sbuf)
        pltpu.sync_copy(dst_ref.at[pl.ds(base + i * IDB, IDB)], dbuf)

        def inner(k, offs2):
            offa, offb = offs2
            s16 = sbuf[pl.ds(k * 16, 16)]
            d16 = dbuf[pl.ds(k * 16, 16)]
            ma = d16 < NHALF
            plsc.store_compressed(lsa.at[pl.ds(offa, 16)], s16, mask=ma)
            plsc.store_compressed(lda.at[pl.ds(offa, 16)], d16, mask=ma)
            mb = jnp.logical_not(ma)
            plsc.store_compressed(lsb.at[pl.ds(offb, 16)], s16, mask=mb)
            plsc.store_compressed(ldb.at[pl.ds(offb, 16)], d16 - NHALF, mask=mb)
            na = plsc.all_reduce_population_count(ma)[0]
            return offa + na, offb + (16 - na)

        return lax.fori_loop(0, IDB // 16, inner, offs)

    offa, offb = lax.fori_loop(0, EPT // IDB, blk_body,
                               (jnp.int32(0), jnp.int32(0)))

    def pad_bucket(off, lsrc, ldst):
        target = ((off + PADC - 1) // PADC) * PADC
        dsv = jnp.full((16,), jnp.int32(N_NODES), jnp.int32)
        ddv = jnp.full((16,), jnp.int32(NHALF), jnp.int32)

        def cond(o):
            return o < target

        def body(o):
            n = jnp.minimum(target - o, 16)
            m = iota < n
            plsc.store_compressed(lsrc.at[pl.ds(o, 16)], dsv, mask=m)
            plsc.store_compressed(ldst.at[pl.ds(o, 16)], ddv, mask=m)
            return o + n

        return lax.while_loop(cond, body, off)

    offa = pad_bucket(offa, lsa, lda)
    offb = pad_bucket(offb, lsb, ldb)

    ra = w * CAPE
    rb = (32 + w) * CAPE
    pltpu.sync_copy(lsa.at[pl.ds(0, CAPE)], bsrc_ref.at[pl.ds(ra, CAPE)])
    pltpu.sync_copy(lda.at[pl.ds(0, CAPE)], bdst_ref.at[pl.ds(ra, CAPE)])
    pltpu.sync_copy(lsb.at[pl.ds(0, CAPE)], bsrc_ref.at[pl.ds(rb, CAPE)])
    pltpu.sync_copy(ldb.at[pl.ds(0, CAPE)], bdst_ref.at[pl.ds(rb, CAPE)])
    ca = jnp.full((16,), offa // CHUNK, jnp.int32)
    cb = jnp.full((16,), offb // CHUNK, jnp.int32)
    lsa[pl.ds(0, 16)] = ca
    pltpu.sync_copy(lsa.at[pl.ds(0, 16)], cnt_ref.at[0, w])
    lsa[pl.ds(0, 16)] = cb
    pltpu.sync_copy(lsa.at[pl.ds(0, 16)], cnt_ref.at[1, w])


@functools.partial(
    pl.kernel,
    out_type=jax.ShapeDtypeStruct((NPAD, HID), jnp.float32),
    mesh=_sc_mesh,
    scratch_types=[
        pltpu.VMEM((2, BIDP, CHUNK), jnp.int32),
        pltpu.VMEM((2, BIDP, CHUNK), jnp.int32),
        pltpu.VMEM((NBP, CHUNK, HID), jnp.float32),
        pltpu.VMEM((16,), jnp.int32),
        pltpu.VMEM_SHARED((ACCR, HID), jnp.float32),
        pltpu.SemaphoreType.DMA((NBP,)),
        pltpu.SemaphoreType.DMA((NBP,)),
        pltpu.SemaphoreType.DMA((2,)),
    ],
    compiler_params=pltpu.CompilerParams(use_tc_tiling_on_sc=False),
)
def _prop64(bsrc_ref, bdst_ref, cnt_ref, q_ref, out_ref, sidx, didx, rows,
            cbuf, acc_sh, gsem, ssem, isem):
    c = lax.axis_index("c")
    s = lax.axis_index("s")
    nt = NHALF // NS
    rb = s * nt
    pltpu.sync_copy(q_ref.at[pl.ds(c * NHALF + rb, nt)],
                    acc_sh.at[pl.ds(rb, nt)])
    plsc.subcore_barrier()
    for roff in range(2):
        r = 2 * s + roff
        pltpu.sync_copy(cnt_ref.at[c, r], cbuf)
        nchunks = cbuf[pl.ds(0, 16)][0]
        _edge_loop(bsrc_ref.at[c, r], bdst_ref.at[c, r], q_ref, acc_sh,
                   sidx, didx, rows, gsem, ssem, isem, 0, nchunks, NBP, BIDP)
    plsc.subcore_barrier()
    pltpu.sync_copy(acc_sh.at[pl.ds(rb, nt)],
                    out_ref.at[pl.ds(c * NHALF + rb, nt)])


def _mm1_body(o1_ref, dinv_ref, w_ref, b_ref, out_ref):
    dinv = dinv_ref[...]
    p = (o1_ref[0] + o1_ref[1]) * dinv
    h = jnp.dot(p, w_ref[...], preferred_element_type=jnp.float32,
                precision=_HI) + b_ref[...]
    out_ref[...] = jnp.maximum(h, 0.0) * dinv


_mm1 = pl.pallas_call(
    _mm1_body,
    grid=(NBLK,),
    in_specs=[
        pl.BlockSpec((NC, BLK, 16), lambda i: (0, i, 0)),
        pl.BlockSpec((BLK, 1), lambda i: (i, 0)),
        pl.BlockSpec((16, HID), lambda i: (0, 0)),
        pl.BlockSpec((1, HID), lambda i: (0, 0)),
    ],
    out_specs=pl.BlockSpec((BLK, HID), lambda i: (i, 0)),
    out_shape=jax.ShapeDtypeStruct((NPAD, HID), jnp.float32),
)


def _mm2_body(o2_ref, dinv_ref, w_ref, b_ref, bat_ref, out_ref, acc, cnt):
    i = pl.program_id(0)

    @pl.when(i == 0)
    def _():
        acc[...] = jnp.zeros_like(acc)
        cnt[...] = jnp.zeros_like(cnt)

    p = o2_ref[...] * dinv_ref[...]
    h = jnp.dot(p, w_ref[...], preferred_element_type=jnp.float32,
                precision=_HI) + b_ref[...]
    h = jnp.maximum(h, 0.0)
    gids = lax.broadcasted_iota(jnp.int32, (1, N_GRAPHS), 1)
    oh = (bat_ref[...] == gids).astype(jnp.float32)
    dn = (((0,), (0,)), ((), ()))
    acc[...] += lax.dot_general(oh, h, dn, preferred_element_type=jnp.float32,
                                precision=_HI)
    cnt[...] += lax.dot_general(oh, jnp.ones((BLK, 1), jnp.float32), dn,
                                preferred_element_type=jnp.float32,
                                precision=_HI)

    @pl.when(i == NBLK - 1)
    def _():
        out_ref[...] = acc[...] / jnp.maximum(cnt[...], 1.0)


_mm2 = pl.pallas_call(
    _mm2_body,
    grid=(NBLK,),
    in_specs=[
        pl.BlockSpec((BLK, HID), lambda i: (i, 0)),
        pl.BlockSpec((BLK, 1), lambda i: (i, 0)),
        pl.BlockSpec((HID, HID), lambda i: (0, 0)),
        pl.BlockSpec((1, HID), lambda i: (0, 0)),
        pl.BlockSpec((BLK, 1), lambda i: (i, 0)),
    ],
    out_specs=pl.BlockSpec((N_GRAPHS, HID), lambda i: (0, 0)),
    out_shape=jax.ShapeDtypeStruct((N_GRAPHS, HID), jnp.float32),
    scratch_shapes=[
        pltpu.VMEM((N_GRAPHS, HID), jnp.float32),
        pltpu.VMEM((N_GRAPHS, 1), jnp.float32),
    ],
)


def kernel(x, edge_index, batch, W1, b1, W2, b2):
    src = edge_index[0].astype(jnp.int32)
    dst = edge_index[1].astype(jnp.int32)
    pad_e = jnp.full((EPAD - N_EDGES,), N_NODES, jnp.int32)
    srcf = jnp.concatenate([src, pad_e])
    dstf = jnp.concatenate([dst, pad_e])
    src2 = srcf.reshape(NCH, CHUNK)
    dst2 = dstf.reshape(NCH, CHUNK)

    degp = _deg(dstf)

    xf = jnp.zeros((NPAD, 16), jnp.float32).at[:N_NODES, :3].set(x)
    dinvf, q1f = _scprep(degp, xf.reshape(NPAD * 16))
    dinv2 = dinvf.reshape(NPAD, 1)
    q1 = q1f.reshape(NPAD, 16)

    z16 = jnp.zeros((NPAD, 16), jnp.float32)
    o1 = _prop16(src2, dst2, q1, z16)

    bsrcf, bdstf, cnts = _bucket(srcf, dstf)
    bsrc4 = bsrcf.reshape(2, 32, CAPC, CHUNK)
    bdst4 = bdstf.reshape(2, 32, CAPC, CHUNK)

    W1p = jnp.zeros((16, HID), jnp.float32).at[:3].set(W1)
    q2 = _mm1(o1, dinv2, W1p, b1.reshape(1, HID))

    o2 = _prop64(bsrc4, bdst4, cnts, q2)

    batp = jnp.concatenate(
        [batch.astype(jnp.int32),
         jnp.full((NPAD - N_NODES,), jnp.int32(2**30), jnp.int32)]
    ).reshape(NPAD, 1)
    out = _mm2(o2, dinv2, W2, b2.reshape(1, HID), batp)
    return out


# speedup vs baseline: 1.7595x; 1.7595x over previous
import functools

import jax
import jax.numpy as jnp
from jax import lax
from jax.experimental import pallas as pl
from jax.experimental.pallas import tpu as pltpu
from jax.experimental.pallas import tpu_sc as plsc

N_NODES = 50000
N_EDGES = 800000
N_GRAPHS = 128
HID = 64

NPAD = 51200
EPAD = 819200
CHUNK = 128
NCH = EPAD // CHUNK
NB16, BID16 = 8, 40
NB32, BID32 = 5, 10
NC = 2
NS = 16
RS = NPAD // NS
BLK = 2048
NBLK = NPAD // BLK

_HI = jax.lax.Precision.DEFAULT

_sc_mesh = plsc.VectorSubcoreMesh(core_axis_name="c", subcore_axis_name="s")


IDB = 1600
EPT = EPAD // (NC * NS)


@functools.partial(
    pl.kernel,
    out_type=jax.ShapeDtypeStruct((NC, NPAD), jnp.float32),
    mesh=_sc_mesh,
    scratch_types=[
        pltpu.VMEM((IDB,), jnp.int32),
        pltpu.VMEM((NPAD,), jnp.float32),
        pltpu.VMEM_SHARED((NS, NPAD), jnp.float32),
        pltpu.VMEM((RS,), jnp.float32),
        pltpu.VMEM((RS,), jnp.float32),
    ],
    compiler_params=pltpu.CompilerParams(needs_layout_passes=False),
)
def _deg(dst_ref, out_ref, ids, degloc, part, tmp, acc):
    c = lax.axis_index("c")
    s = lax.axis_index("s")
    w = c * NS + s
    z16 = jnp.zeros((16,), jnp.float32)

    def zero_body(i, carry):
        for j in range(4):
            degloc[pl.ds((i * 4 + j) * 16, 16)] = z16
        return carry

    lax.fori_loop(0, NPAD // 64, zero_body, 0)

    ones = jnp.ones((16,), jnp.float32)
    base = w * EPT

    def blk_body(i, carry):
        pltpu.sync_copy(dst_ref.at[pl.ds(base + i * IDB, IDB)], ids)

        def inner(k, carry2):
            for j in range(4):
                idx = ids[pl.ds((k * 4 + j) * 16, 16)]
                plsc.addupdate_scatter(degloc, [idx], ones)
            return carry2

        lax.fori_loop(0, IDB // 64, inner, 0)
        return carry

    lax.fori_loop(0, EPT // IDB, blk_body, 0)

    pltpu.sync_copy(degloc, part.at[s])
    plsc.subcore_barrier()
    rb = s * RS
    pltpu.sync_copy(part.at[0, pl.ds(rb, RS)], acc)

    def red_body(t, carry):
        pltpu.sync_copy(part.at[t, pl.ds(rb, RS)], tmp)

        def vadd(i, carry2):
            for j in range(4):
                o = (i * 4 + j) * 16
                acc[pl.ds(o, 16)] = acc[pl.ds(o, 16)] + tmp[pl.ds(o, 16)]
            return carry2

        lax.fori_loop(0, RS // 64, vadd, 0)
        return carry

    lax.fori_loop(1, NS, red_body, 0)
    pltpu.sync_copy(acc, out_ref.at[c, pl.ds(rb, RS)])


NT = NPAD // (NC * NS)


def _rsqrt16(x):
    i = plsc.bitcast(x, jnp.int32)
    y = plsc.bitcast(jnp.int32(0x5F3759DF) - (i >> 1), jnp.float32)
    y = y * (1.5 - 0.5 * x * y * y)
    y = y * (1.5 - 0.5 * x * y * y)
    y = y * (1.5 - 0.5 * x * y * y)
    return y


@functools.partial(
    pl.kernel,
    out_type=[
        jax.ShapeDtypeStruct((NPAD,), jnp.float32),
        jax.ShapeDtypeStruct((NPAD * 16,), jnp.float32),
    ],
    mesh=_sc_mesh,
    scratch_types=[
        pltpu.VMEM((NT,), jnp.float32),
        pltpu.VMEM((NT,), jnp.float32),
        pltpu.VMEM((NT,), jnp.float32),
        pltpu.VMEM((NT * 16,), jnp.float32),
    ],
    compiler_params=pltpu.CompilerParams(use_tc_tiling_on_sc=False,
                                         needs_layout_passes=False),
)
def _scprep(degp_ref, xf_ref, dinv_ref, q1_ref, da, db, dloc, xbuf):
    c = lax.axis_index("c")
    s = lax.axis_index("s")
    nb = (c * NS + s) * NT
    pltpu.sync_copy(degp_ref.at[0, pl.ds(nb, NT)], da)
    pltpu.sync_copy(degp_ref.at[1, pl.ds(nb, NT)], db)
    pltpu.sync_copy(xf_ref.at[pl.ds(nb * 16, NT * 16)], xbuf)

    def vec_body(i, carry):
        d = da[pl.ds(i * 16, 16)] + db[pl.ds(i * 16, 16)] + 1.0
        dloc[pl.ds(i * 16, 16)] = _rsqrt16(d)
        return carry

    lax.fori_loop(0, NT // 16, vec_body, 0)
    pltpu.sync_copy(dloc, dinv_ref.at[pl.ds(nb, NT)])

    def row_body(i, carry):
        dv = dloc[pl.ds(i * 16, 16)]
        for k in range(16):
            base = (i * 16 + k) * 16
            xbuf[pl.ds(base, 16)] = xbuf[pl.ds(base, 16)] * dv[k]
        return carry

    lax.fori_loop(0, NT // 16, row_body, 0)
    pltpu.sync_copy(xbuf, q1_ref.at[pl.ds(nb * 16, NT * 16)])


def _edge_loop(src_ref, dst_ref, gather_ref, acc_sh, sidx, didx, rows, gsem,
               ssem, isem, c0, ntile, nb, bid):
    nblk = ntile // bid

    def load_ids(ib, blk):
        cb = c0 + blk * bid
        pltpu.async_copy(src_ref.at[pl.ds(cb, bid)], sidx.at[ib], isem.at[ib])
        pltpu.async_copy(dst_ref.at[pl.ds(cb, bid)], didx.at[ib], isem.at[ib])

    def wait_ids(ib):
        for r in (sidx, didx):
            pltpu.make_async_copy(src_ref.at[pl.ds(0, bid)], r.at[ib],
                                  isem.at[ib]).wait()

    def fire_gather(ib, u, b):
        pltpu.async_copy(gather_ref.at[sidx.at[ib, u]], rows.at[b],
                         gsem.at[b])

    def drain_gather(b):
        pltpu.make_async_copy(gather_ref.at[pl.ds(0, CHUNK)], rows.at[b],
                              gsem.at[b]).wait()

    def fire_add(ib, u, b):
        pltpu.async_copy(rows.at[b], acc_sh.at[didx.at[ib, u]], ssem.at[b],
                         add=True)

    def drain_add(b):
        pltpu.make_async_copy(rows.at[b], acc_sh.at[pl.ds(0, CHUNK)],
                              ssem.at[b]).wait()

    load_ids(0, 0)

    def blk_body(blk, carry):
        ib = blk % 2
        wait_ids(ib)

        @pl.when(blk + 1 < nblk)
        def _():
            load_ids(1 - ib, blk + 1)

        for u in range(nb - 1):
            fire_gather(ib, u, u)
        for u in range(bid):
            b = u % nb
            drain_gather(b)
            fire_add(ib, u, b)
            nxt = u + nb - 1
            if nxt < bid:
                pb = nxt % nb
                if u >= 1:
                    drain_add(pb)
                fire_gather(ib, nxt, pb)
        for b in range(nb):
            drain_add(b)
        return carry

    lax.fori_loop(0, nblk, blk_body, 0)


@functools.partial(
    pl.kernel,
    out_type=jax.ShapeDtypeStruct((NC, NPAD, 16), jnp.float32),
    mesh=_sc_mesh,
    scratch_types=[
        pltpu.VMEM((2, BID16, CHUNK), jnp.int32),
        pltpu.VMEM((2, BID16, CHUNK), jnp.int32),
        pltpu.VMEM((NB16, CHUNK, 16), jnp.float32),
        pltpu.VMEM_SHARED((NPAD, 16), jnp.float32),
        pltpu.SemaphoreType.DMA((NB16,)),
        pltpu.SemaphoreType.DMA((NB16,)),
        pltpu.SemaphoreType.DMA((2,)),
    ],
    compiler_params=pltpu.CompilerParams(use_tc_tiling_on_sc=False),
)
def _prop16(src_ref, dst_ref, q_ref, z_ref, out_ref, sidx, didx, rows, acc_sh,
            gsem, ssem, isem):
    c = lax.axis_index("c")
    s = lax.axis_index("s")
    rb = s * RS

    @pl.when(c == 0)
    def _():
        pltpu.sync_copy(q_ref.at[pl.ds(rb, RS)], acc_sh.at[pl.ds(rb, RS)])

    @pl.when(c == 1)
    def _():
        pltpu.sync_copy(z_ref.at[pl.ds(rb, RS)], acc_sh.at[pl.ds(rb, RS)])

    plsc.subcore_barrier()
    per_core = NCH // NC
    per_tile = per_core // NS
    c0 = c * per_core + s * per_tile
    _edge_loop(src_ref, dst_ref, q_ref, acc_sh, sidx, didx, rows, gsem,
               ssem, isem, c0, per_tile, NB16, BID16)
    plsc.subcore_barrier()
    pltpu.sync_copy(acc_sh.at[pl.ds(rb, RS)], out_ref.at[c, pl.ds(rb, RS)])


@functools.partial(
    pl.kernel,
    out_type=jax.ShapeDtypeStruct((NC, NPAD, 32), jnp.float32),
    mesh=_sc_mesh,
    scratch_types=[
        pltpu.VMEM((2, BID32, CHUNK), jnp.int32),
        pltpu.VMEM((2, BID32, CHUNK), jnp.int32),
        pltpu.VMEM((NB32, CHUNK, 32), jnp.float32),
        pltpu.VMEM_SHARED((NPAD, 32), jnp.float32),
        pltpu.SemaphoreType.DMA((NB32,)),
        pltpu.SemaphoreType.DMA((NB32,)),
        pltpu.SemaphoreType.DMA((2,)),
    ],
    compiler_params=pltpu.CompilerParams(use_tc_tiling_on_sc=False),
)
def _prop32(src_ref, dst_ref, q_ref, out_ref, sidx, didx, rows, acc_sh, gsem,
            ssem, isem):
    c = lax.axis_index("c")
    s = lax.axis_index("s")
    rb = s * RS
    pltpu.sync_copy(q_ref.at[c, pl.ds(rb, RS)], acc_sh.at[pl.ds(rb, RS)])
    plsc.subcore_barrier()
    per_tile = NCH // NS
    c0 = s * per_tile
    _edge_loop(src_ref, dst_ref, q_ref.at[c], acc_sh, sidx, didx, rows, gsem,
               ssem, isem, c0, per_tile, NB32, BID32)
    plsc.subcore_barrier()
    pltpu.sync_copy(acc_sh.at[pl.ds(rb, RS)], out_ref.at[c, pl.ds(rb, RS)])


def _mm1_body(o1_ref, dinv_ref, w_ref, b_ref, out_ref):
    dinv = dinv_ref[...]
    p = (o1_ref[0] + o1_ref[1]) * dinv
    h = jnp.dot(p, w_ref[...], preferred_element_type=jnp.float32,
                precision=_HI) + b_ref[...]
    g = jnp.maximum(h, 0.0) * dinv
    out_ref[0] = g[:, :32]
    out_ref[1] = g[:, 32:]


_mm1 = pl.pallas_call(
    _mm1_body,
    grid=(NBLK,),
    in_specs=[
        pl.BlockSpec((NC, BLK, 16), lambda i: (0, i, 0)),
        pl.BlockSpec((BLK, 1), lambda i: (i, 0)),
        pl.BlockSpec((16, HID), lambda i: (0, 0)),
        pl.BlockSpec((1, HID), lambda i: (0, 0)),
    ],
    out_specs=pl.BlockSpec((NC, BLK, 32), lambda i: (0, i, 0)),
    out_shape=jax.ShapeDtypeStruct((NC, NPAD, 32), jnp.float32),
)


def _mm2_body(o2_ref, dinv_ref, w_ref, b_ref, bat_ref, out_ref, acc, cnt):
    i = pl.program_id(0)

    @pl.when(i == 0)
    def _():
        acc[...] = jnp.zeros_like(acc)
        cnt[...] = jnp.zeros_like(cnt)

    p = jnp.concatenate([o2_ref[0], o2_ref[1]], axis=1) * dinv_ref[...]
    h = jnp.dot(p, w_ref[...], preferred_element_type=jnp.float32,
                precision=_HI) + b_ref[...]
    h = jnp.maximum(h, 0.0)
    gids = lax.broadcasted_iota(jnp.int32, (1, N_GRAPHS), 1)
    oh = (bat_ref[...] == gids).astype(jnp.float32)
    dn = (((0,), (0,)), ((), ()))
    acc[...] += lax.dot_general(oh, h, dn, preferred_element_type=jnp.float32,
                                precision=_HI)
    cnt[...] += lax.dot_general(oh, jnp.ones((BLK, 1), jnp.float32), dn,
                                preferred_element_type=jnp.float32,
                                precision=_HI)

    @pl.when(i == NBLK - 1)
    def _():
        out_ref[...] = acc[...] / jnp.maximum(cnt[...], 1.0)


_mm2 = pl.pallas_call(
    _mm2_body,
    grid=(NBLK,),
    in_specs=[
        pl.BlockSpec((NC, BLK, 32), lambda i: (0, i, 0)),
        pl.BlockSpec((BLK, 1), lambda i: (i, 0)),
        pl.BlockSpec((HID, HID), lambda i: (0, 0)),
        pl.BlockSpec((1, HID), lambda i: (0, 0)),
        pl.BlockSpec((BLK, 1), lambda i: (i, 0)),
    ],
    out_specs=pl.BlockSpec((N_GRAPHS, HID), lambda i: (0, 0)),
    out_shape=jax.ShapeDtypeStruct((N_GRAPHS, HID), jnp.float32),
    scratch_shapes=[
        pltpu.VMEM((N_GRAPHS, HID), jnp.float32),
        pltpu.VMEM((N_GRAPHS, 1), jnp.float32),
    ],
)


def kernel(x, edge_index, batch, W1, b1, W2, b2):
    src = edge_index[0].astype(jnp.int32)
    dst = edge_index[1].astype(jnp.int32)
    pad_e = jnp.full((EPAD - N_EDGES,), N_NODES, jnp.int32)
    srcf = jnp.concatenate([src, pad_e])
    dstf = jnp.concatenate([dst, pad_e])
    src2 = srcf.reshape(NCH, CHUNK)
    dst2 = dstf.reshape(NCH, CHUNK)

    degp = _deg(dstf)

    xf = jnp.zeros((NPAD, 16), jnp.float32).at[:N_NODES, :3].set(x)
    dinvf, q1f = _scprep(degp, xf.reshape(NPAD * 16))
    dinv2 = dinvf.reshape(NPAD, 1)
    q1 = q1f.reshape(NPAD, 16)

    z16 = jnp.zeros((NPAD, 16), jnp.float32)
    o1 = _prop16(src2, dst2, q1, z16)

    W1p = jnp.zeros((16, HID), jnp.float32).at[:3].set(W1)
    q2 = _mm1(o1, dinv2, W1p, b1.reshape(1, HID))

    o2 = _prop32(src2, dst2, q2)

    batp = jnp.concatenate(
        [batch.astype(jnp.int32),
         jnp.full((NPAD - N_NODES,), jnp.int32(2**30), jnp.int32)]
    ).reshape(NPAD, 1)
    out = _mm2(o2, dinv2, W2, b2.reshape(1, HID), batp)
    return out
